# Initial kernel scaffold; baseline (speedup 1.0000x reference)
import functools

import jax
import jax.numpy as jnp
from jax import lax
from jax.experimental import pallas as pl
from jax.experimental.pallas import tpu as pltpu
from jax.experimental.pallas import tpu_sc as plsc

NR = 50000
DIN = 128
H = 64
HH = 32
OUT = 32
E = 400000
NT = 16
TPR = 3136
PAD = NT * TPR
EPT = E // NT
EC = 1000
NCH = EPT // EC
DR = 784
NDR = TPR // DR
BB = 2000
NB = NR // BB

_mesh = plsc.VectorSubcoreMesh(core_axis_name="c", subcore_axis_name="s")


def _counts_body(dst_p_hbm, dst_r_hbm, ones_hbm, zrows_hbm,
                 cnt_p_hbm, cnt_r_hbm,
                 dst_v, ones_v, zstage_v, acc_sh):
    c = lax.axis_index("c")
    s = lax.axis_index("s")
    pltpu.sync_copy(ones_hbm, ones_v)
    pltpu.sync_copy(zrows_hbm, zstage_v)
    for j in range(NDR):
        pltpu.sync_copy(zstage_v, acc_sh.at[pl.ds(s * TPR + j * DR, DR)])
    plsc.subcore_barrier()

    def chunk(k, dst_hbm):
        base = s * EPT + k * EC
        pltpu.sync_copy(dst_hbm.at[pl.ds(base, EC)], dst_v)
        pltpu.sync_copy(ones_v, acc_sh.at[dst_v], add=True)
        return dst_hbm

    @pl.when(c == 0)
    def _():
        lax.fori_loop(0, NCH, chunk, dst_p_hbm)

    @pl.when(c == 1)
    def _():
        lax.fori_loop(0, NCH, chunk, dst_r_hbm)

    plsc.subcore_barrier()
    for j in range(NDR):
        rows = pl.ds(s * TPR + j * DR, DR)

        @pl.when(c == 0)
        def _():
            pltpu.sync_copy(acc_sh.at[rows], cnt_p_hbm.at[rows])

        @pl.when(c == 1)
        def _():
            pltpu.sync_copy(acc_sh.at[rows], cnt_r_hbm.at[rows])


_counts_call = functools.partial(
    pl.kernel,
    out_type=[jax.ShapeDtypeStruct((PAD, 16), jnp.float32),
              jax.ShapeDtypeStruct((PAD, 16), jnp.float32)],
    mesh=_mesh,
    scratch_types=[
        pltpu.VMEM((EC,), jnp.int32),
        pltpu.VMEM((EC, 16), jnp.float32),
        pltpu.VMEM((DR, 16), jnp.float32),
        pltpu.VMEM_SHARED((PAD, 16), jnp.float32),
    ],
)(_counts_body)


def _agg_body(rev0_hbm, rev1_hbm, prod0_hbm, prod1_hbm,
              src_r_hbm, dst_p_hbm, src_p_hbm, dst_r_hbm, zrows_hbm,
              aggp0_hbm, aggp1_hbm, aggr0_hbm, aggr1_hbm,
              src_v, dst_v, rows_v, zstage_v, acc_sh, sem):
    c = lax.axis_index("c")
    s = lax.axis_index("s")
    pltpu.sync_copy(zrows_hbm, zstage_v)

    def one_relation(src_hbm, dst_hbm, tab0_hbm, tab1_hbm, out0_hbm, out1_hbm):
        for j in range(NDR):
            pltpu.sync_copy(zstage_v, acc_sh.at[pl.ds(s * TPR + j * DR, DR)])
        plsc.subcore_barrier()

        def chunk(k, carry):
            base = s * EPT + k * EC
            pltpu.sync_copy(src_hbm.at[pl.ds(base, EC)], src_v)
            pltpu.sync_copy(dst_hbm.at[pl.ds(base, EC)], dst_v)

            @pl.when(c == 0)
            def _():
                pltpu.async_copy(tab0_hbm.at[src_v], rows_v, sem).wait()

            @pl.when(c == 1)
            def _():
                pltpu.async_copy(tab1_hbm.at[src_v], rows_v, sem).wait()

            pltpu.sync_copy(rows_v, acc_sh.at[dst_v], add=True)
            return carry

        lax.fori_loop(0, NCH, chunk, 0)
        plsc.subcore_barrier()
        for j in range(NDR):
            rows = pl.ds(s * TPR + j * DR, DR)

            @pl.when(c == 0)
            def _():
                pltpu.sync_copy(acc_sh.at[rows], out0_hbm.at[rows])

            @pl.when(c == 1)
            def _():
                pltpu.sync_copy(acc_sh.at[rows], out1_hbm.at[rows])

    one_relation(src_r_hbm, dst_p_hbm, rev0_hbm, rev1_hbm, aggp0_hbm, aggp1_hbm)
    one_relation(src_p_hbm, dst_r_hbm, prod0_hbm, prod1_hbm, aggr0_hbm, aggr1_hbm)


_agg_call = functools.partial(
    pl.kernel,
    out_type=[jax.ShapeDtypeStruct((PAD, HH), jnp.float32)] * 4,
    mesh=_mesh,
    scratch_types=[
        pltpu.VMEM((EC,), jnp.int32),
        pltpu.VMEM((EC,), jnp.int32),
        pltpu.VMEM((EC, HH), jnp.float32),
        pltpu.VMEM((DR, HH), jnp.float32),
        pltpu.VMEM_SHARED((PAD, HH), jnp.float32),
        pltpu.SemaphoreType.DMA,
    ],
)(_agg_body)


def _proj_body(xr_ref, xp_ref, wr_ref, br_ref, wp_ref, bp_ref,
               r0_ref, r1_ref, p0_ref, p1_ref):
    hr = jnp.dot(xr_ref[...], wr_ref[...],
                 preferred_element_type=jnp.float32) + br_ref[...]
    hp = jnp.dot(xp_ref[...], wp_ref[...],
                 preferred_element_type=jnp.float32) + bp_ref[...]
    r0_ref[...] = hr[:, :HH]
    r1_ref[...] = hr[:, HH:]
    p0_ref[...] = hp[:, :HH]
    p1_ref[...] = hp[:, HH:]


def _proj(x_review, x_product, W_review, b_review, W_product, b_product):
    blk = lambda: pl.BlockSpec((BB, DIN), lambda i: (i, 0))
    full = lambda shape: pl.BlockSpec(shape, lambda i: tuple(0 for _ in shape))
    outb = lambda: pl.BlockSpec((BB, HH), lambda i: (i, 0))
    return pl.pallas_call(
        _proj_body,
        grid=(NB,),
        in_specs=[blk(), blk(), full((DIN, H)), full((1, H)),
                  full((DIN, H)), full((1, H))],
        out_specs=[outb() for _ in range(4)],
        out_shape=[jax.ShapeDtypeStruct((NR, HH), jnp.float32)] * 4,
    )(x_review, x_product, W_review, b_review.reshape(1, H),
      W_product, b_product.reshape(1, H))


def _combine_body(split_out, relu,
                  hr0_ref, hr1_ref, hp0_ref, hp1_ref,
                  ap0_ref, ap1_ref, ar0_ref, ar1_ref,
                  cp_ref, cr_ref, wroot_ref, wrel_ref, b_ref,
                  *out_refs):
    inv_p = 1.0 / jnp.maximum(cp_ref[:, 0:1], 1.0)
    inv_r = 1.0 / jnp.maximum(cr_ref[:, 0:1], 1.0)
    h_rev = jnp.concatenate([hr0_ref[...], hr1_ref[...]], axis=1)
    h_prod = jnp.concatenate([hp0_ref[...], hp1_ref[...]], axis=1)
    agg_p = jnp.concatenate([ap0_ref[...], ap1_ref[...]], axis=1) * inv_p
    agg_r = jnp.concatenate([ar0_ref[...], ar1_ref[...]], axis=1) * inv_r
    wroot = wroot_ref[...]
    b = b_ref[...]
    out_rev = (jnp.dot(h_rev, wroot, preferred_element_type=jnp.float32) + b
               + jnp.dot(agg_r, wrel_ref[1], preferred_element_type=jnp.float32))
    out_prod = (jnp.dot(h_prod, wroot, preferred_element_type=jnp.float32) + b
                + jnp.dot(agg_p, wrel_ref[0], preferred_element_type=jnp.float32))
    if relu:
        out_rev = jnp.maximum(out_rev, 0.0)
        out_prod = jnp.maximum(out_prod, 0.0)
    if split_out:
        out_refs[0][...] = out_rev[:, :HH]
        out_refs[1][...] = out_rev[:, HH:]
        out_refs[2][...] = out_prod[:, :HH]
        out_refs[3][...] = out_prod[:, HH:]
    else:
        out_refs[0][0] = out_rev
        out_refs[0][1] = out_prod


def _combine(planes, aggs, cnts, w_rel, w_root, b, split_out, relu):
    dout = w_root.shape[1]
    nblk = lambda w: pl.BlockSpec((BB, w), lambda i: (i, 0))
    full = lambda shape: pl.BlockSpec(shape, lambda i: tuple(0 for _ in shape))
    if split_out:
        out_specs = [nblk(HH) for _ in range(4)]
        out_shape = [jax.ShapeDtypeStruct((NR, HH), jnp.float32)] * 4
    else:
        out_specs = [pl.BlockSpec((2, BB, dout), lambda i: (0, i, 0))]
        out_shape = [jax.ShapeDtypeStruct((2, NR, dout), jnp.float32)]
    return pl.pallas_call(
        functools.partial(_combine_body, split_out, relu),
        grid=(NB,),
        in_specs=[nblk(HH) for _ in range(8)] + [nblk(16) for _ in range(2)]
                 + [full((H, dout)), full((2, H, dout)), full((1, dout))],
        out_specs=out_specs,
        out_shape=out_shape,
    )(*planes, *aggs, *cnts, w_root, w_rel, b.reshape(1, dout))


def kernel(x_review, x_product, edge_index_r2p, edge_index_p2r,
           W_review, b_review, W_product, b_product,
           conv1_w_rel, conv1_w_root, conv1_b,
           conv2_w_rel, conv2_w_root, conv2_b):
    src_r = edge_index_r2p[0]
    dst_p = edge_index_r2p[1] - NR
    src_p = edge_index_p2r[0] - NR
    dst_r = edge_index_p2r[1]
    ones16 = jnp.ones((EC, 16), jnp.float32)
    zrows = jnp.zeros((DR, HH), jnp.float32)
    zrows16 = jnp.zeros((DR, 16), jnp.float32)

    cnt_p, cnt_r = _counts_call(dst_p, dst_r, ones16, zrows16)
    planes1 = _proj(x_review, x_product, W_review, b_review,
                    W_product, b_product)
    aggs1 = _agg_call(*planes1, src_r, dst_p, src_p, dst_r, zrows)
    planes2 = _combine(planes1, aggs1, (cnt_p, cnt_r),
                       conv1_w_rel, conv1_w_root, conv1_b,
                       split_out=True, relu=True)
    aggs2 = _agg_call(*planes2, src_r, dst_p, src_p, dst_r, zrows)
    (out,) = _combine(planes2, aggs2, (cnt_p, cnt_r),
                      conv2_w_rel, conv2_w_root, conv2_b,
                      split_out=False, relu=False)
    return out.reshape(2 * NR, OUT)


# speedup vs baseline: 3.4305x; 3.4305x over previous
import functools

import jax
import jax.numpy as jnp
from jax import lax
from jax.experimental import pallas as pl
from jax.experimental.pallas import tpu as pltpu
from jax.experimental.pallas import tpu_sc as plsc

NR = 50000
DIN = 128
H = 64
HH = 32
OUT = 32
E = 400000
NT = 16
TPR = 3136
PAD = NT * TPR
EPT = E // NT
EC = 200
NCH = EPT // EC
DR = 196
NDR = TPR // DR
BB = 2000
NB = NR // BB

_mesh = plsc.VectorSubcoreMesh(core_axis_name="c", subcore_axis_name="s")


def _sc_body(rev0_hbm, rev1_hbm, prod0_hbm, prod1_hbm,
             src_r_hbm, dst_p_hbm, src_p_hbm, dst_r_hbm,
             ones_hbm, zrows_hbm,
             cnt_p_hbm, cnt_r_hbm,
             aggp0_hbm, aggp1_hbm, aggr0_hbm, aggr1_hbm,
             src_v, dst_v, rows_v, zstage_v, acc_sh, sem):
    c = lax.axis_index("c")
    s = lax.axis_index("s")
    pltpu.sync_copy(zrows_hbm, zstage_v)
    pltpu.sync_copy(ones_hbm, rows_v)

    def zero_own_slice():
        for j in range(NDR):
            pltpu.sync_copy(zstage_v, acc_sh.at[pl.ds(s * TPR + j * DR, DR)])

    zero_own_slice()
    plsc.subcore_barrier()

    def count_loop(dst_hbm):
        def chunk(k, carry):
            base = s * EPT + k * EC
            pltpu.sync_copy(dst_hbm.at[pl.ds(base, EC)], dst_v)
            pltpu.sync_copy(rows_v, acc_sh.at[dst_v], add=True)
            return carry
        lax.fori_loop(0, NCH, chunk, 0)

    @pl.when(c == 0)
    def _():
        count_loop(dst_p_hbm)

    @pl.when(c == 1)
    def _():
        count_loop(dst_r_hbm)

    plsc.subcore_barrier()
    for j in range(NDR):
        rows = pl.ds(s * TPR + j * DR, DR)

        @pl.when(c == 0)
        def _():
            pltpu.sync_copy(acc_sh.at[rows], cnt_p_hbm.at[rows])

        @pl.when(c == 1)
        def _():
            pltpu.sync_copy(acc_sh.at[rows], cnt_r_hbm.at[rows])

    def one_relation(src_hbm, dst_hbm, tab0_hbm, tab1_hbm, out0_hbm, out1_hbm):
        zero_own_slice()
        plsc.subcore_barrier()

        def chunk(k, carry):
            base = s * EPT + k * EC
            pltpu.sync_copy(src_hbm.at[pl.ds(base, EC)], src_v)
            pltpu.sync_copy(dst_hbm.at[pl.ds(base, EC)], dst_v)

            @pl.when(c == 0)
            def _():
                pltpu.async_copy(tab0_hbm.at[src_v], rows_v, sem).wait()

            @pl.when(c == 1)
            def _():
                pltpu.async_copy(tab1_hbm.at[src_v], rows_v, sem).wait()

            pltpu.sync_copy(rows_v, acc_sh.at[dst_v], add=True)
            return carry

        lax.fori_loop(0, NCH, chunk, 0)
        plsc.subcore_barrier()
        for j in range(NDR):
            rows = pl.ds(s * TPR + j * DR, DR)

            @pl.when(c == 0)
            def _():
                pltpu.sync_copy(acc_sh.at[rows], out0_hbm.at[rows])

            @pl.when(c == 1)
            def _():
                pltpu.sync_copy(acc_sh.at[rows], out1_hbm.at[rows])

    one_relation(src_r_hbm, dst_p_hbm, rev0_hbm, rev1_hbm, aggp0_hbm, aggp1_hbm)
    one_relation(src_p_hbm, dst_r_hbm, prod0_hbm, prod1_hbm, aggr0_hbm, aggr1_hbm)


_sc_call = functools.partial(
    pl.kernel,
    out_type=[jax.ShapeDtypeStruct((PAD, HH), jnp.float32)] * 6,
    mesh=_mesh,
    scratch_types=[
        pltpu.VMEM((EC,), jnp.int32),
        pltpu.VMEM((EC,), jnp.int32),
        pltpu.VMEM((EC, HH), jnp.float32),
        pltpu.VMEM((DR, HH), jnp.float32),
        pltpu.VMEM_SHARED((PAD, HH), jnp.float32),
        pltpu.SemaphoreType.DMA,
    ],
    compiler_params=pltpu.CompilerParams(use_tc_tiling_on_sc=False),
)(_sc_body)


def _proj_body(xr_ref, xp_ref, wr_ref, br_ref, wp_ref, bp_ref,
               r0_ref, r1_ref, p0_ref, p1_ref):
    hr = jnp.dot(xr_ref[...], wr_ref[...],
                 preferred_element_type=jnp.float32) + br_ref[...]
    hp = jnp.dot(xp_ref[...], wp_ref[...],
                 preferred_element_type=jnp.float32) + bp_ref[...]
    r0_ref[...] = hr[:, :HH]
    r1_ref[...] = hr[:, HH:]
    p0_ref[...] = hp[:, :HH]
    p1_ref[...] = hp[:, HH:]


def _proj(x_review, x_product, W_review, b_review, W_product, b_product):
    blk = lambda: pl.BlockSpec((BB, DIN), lambda i: (i, 0))
    full = lambda shape: pl.BlockSpec(shape, lambda i: tuple(0 for _ in shape))
    outb = lambda: pl.BlockSpec((BB, HH), lambda i: (i, 0))
    return pl.pallas_call(
        _proj_body,
        grid=(NB,),
        in_specs=[blk(), blk(), full((DIN, H)), full((1, H)),
                  full((DIN, H)), full((1, H))],
        out_specs=[outb() for _ in range(4)],
        out_shape=[jax.ShapeDtypeStruct((NR, HH), jnp.float32)] * 4,
    )(x_review, x_product, W_review, b_review.reshape(1, H),
      W_product, b_product.reshape(1, H))


def _combine_body(split_out, relu,
                  hr0_ref, hr1_ref, hp0_ref, hp1_ref,
                  ap0_ref, ap1_ref, ar0_ref, ar1_ref,
                  cp_ref, cr_ref, wroot_ref, wrel_ref, b_ref,
                  *out_refs):
    inv_p = 1.0 / jnp.maximum(cp_ref[:, 0:1], 1.0)
    inv_r = 1.0 / jnp.maximum(cr_ref[:, 0:1], 1.0)
    h_rev = jnp.concatenate([hr0_ref[...], hr1_ref[...]], axis=1)
    h_prod = jnp.concatenate([hp0_ref[...], hp1_ref[...]], axis=1)
    agg_p = jnp.concatenate([ap0_ref[...], ap1_ref[...]], axis=1) * inv_p
    agg_r = jnp.concatenate([ar0_ref[...], ar1_ref[...]], axis=1) * inv_r
    wroot = wroot_ref[...]
    b = b_ref[...]
    out_rev = (jnp.dot(h_rev, wroot, preferred_element_type=jnp.float32) + b
               + jnp.dot(agg_r, wrel_ref[1], preferred_element_type=jnp.float32))
    out_prod = (jnp.dot(h_prod, wroot, preferred_element_type=jnp.float32) + b
                + jnp.dot(agg_p, wrel_ref[0], preferred_element_type=jnp.float32))
    if relu:
        out_rev = jnp.maximum(out_rev, 0.0)
        out_prod = jnp.maximum(out_prod, 0.0)
    if split_out:
        out_refs[0][...] = out_rev[:, :HH]
        out_refs[1][...] = out_rev[:, HH:]
        out_refs[2][...] = out_prod[:, :HH]
        out_refs[3][...] = out_prod[:, HH:]
    else:
        out_refs[0][0] = out_rev
        out_refs[0][1] = out_prod


def _combine(planes, aggs, cnts, w_rel, w_root, b, split_out, relu):
    dout = w_root.shape[1]
    nblk = lambda w: pl.BlockSpec((BB, w), lambda i: (i, 0))
    full = lambda shape: pl.BlockSpec(shape, lambda i: tuple(0 for _ in shape))
    if split_out:
        out_specs = [nblk(HH) for _ in range(4)]
        out_shape = [jax.ShapeDtypeStruct((NR, HH), jnp.float32)] * 4
    else:
        out_specs = [pl.BlockSpec((2, BB, dout), lambda i: (0, i, 0))]
        out_shape = [jax.ShapeDtypeStruct((2, NR, dout), jnp.float32)]
    return pl.pallas_call(
        functools.partial(_combine_body, split_out, relu),
        grid=(NB,),
        in_specs=[nblk(HH) for _ in range(8)] + [nblk(HH) for _ in range(2)]
                 + [full((H, dout)), full((2, H, dout)), full((1, dout))],
        out_specs=out_specs,
        out_shape=out_shape,
    )(*planes, *aggs, *cnts, w_root, w_rel, b.reshape(1, dout))


def kernel(x_review, x_product, edge_index_r2p, edge_index_p2r,
           W_review, b_review, W_product, b_product,
           conv1_w_rel, conv1_w_root, conv1_b,
           conv2_w_rel, conv2_w_root, conv2_b):
    src_r = edge_index_r2p[0]
    dst_p = edge_index_r2p[1] - NR
    src_p = edge_index_p2r[0] - NR
    dst_r = edge_index_p2r[1]
    ones = jnp.ones((EC, HH), jnp.float32)
    zrows = jnp.zeros((DR, HH), jnp.float32)

    planes1 = _proj(x_review, x_product, W_review, b_review,
                    W_product, b_product)
    cnt_p, cnt_r, *aggs1 = _sc_call(*planes1, src_r, dst_p, src_p, dst_r,
                                    ones, zrows)
    planes2 = _combine(planes1, aggs1, (cnt_p, cnt_r),
                       conv1_w_rel, conv1_w_root, conv1_b,
                       split_out=True, relu=True)
    cnt_p2, cnt_r2, *aggs2 = _sc_call(*planes2, src_r, dst_p, src_p, dst_r,
                                      ones, zrows)
    (out,) = _combine(planes2, aggs2, (cnt_p, cnt_r),
                      conv2_w_rel, conv2_w_root, conv2_b,
                      split_out=False, relu=False)
    return out.reshape(2 * NR, OUT)


# speedup vs baseline: 5.7427x; 1.6740x over previous
import functools

import jax
import jax.numpy as jnp
from jax import lax
from jax.experimental import pallas as pl
from jax.experimental.pallas import tpu as pltpu
from jax.experimental.pallas import tpu_sc as plsc

NR = 50000
DIN = 128
H = 64
HH = 32
OUT = 32
E = 400000
NT = 16
TPR = 3136
PAD = NT * TPR
EPT = E // NT
EC = 200
NCH = EPT // EC
NCB = 5
NBLK = NCH // NCB
DR = 196
NDR = TPR // DR
BB = 2000
NB = NR // BB

_mesh = plsc.VectorSubcoreMesh(core_axis_name="c", subcore_axis_name="s")


def _sc_body(tabr3_hbm, tabp3_hbm,
             src_r2_hbm, dst_p2_hbm, src_p2_hbm, dst_r2_hbm,
             ones_hbm, zrows_hbm, flag_hbm,
             cnt3_hbm, aggp3_hbm, aggr3_hbm,
             idxs_v, idxd_v, r0_v, r1_v, r2_v, zstage_v, flag_v, acc_sh,
             g0, g1, g2, s0, s1, s2, zsem):
    c = lax.axis_index("c")
    s = lax.axis_index("s")
    rows = [r0_v, r1_v, r2_v]
    gsem = [g0, g1, g2]
    ssem = [s0, s1, s2]
    pltpu.sync_copy(zrows_hbm, zstage_v)
    pltpu.sync_copy(flag_hbm, flag_v)
    do_counts = jnp.max(flag_v[...])

    def zero_own():
        cps = [pltpu.async_copy(
                   zstage_v, acc_sh.at[pl.ds(s * TPR + j * DR, DR)], zsem)
               for j in range(NDR)]
        for cp in cps:
            cp.wait()

    def drain_own(out2):
        cps = [pltpu.async_copy(
                   acc_sh.at[pl.ds(s * TPR + j * DR, DR)],
                   out2.at[pl.ds(s * TPR + j * DR, DR)], zsem)
               for j in range(NDR)]
        for cp in cps:
            cp.wait()

    @pl.when(do_counts > 0)
    def _():
        pltpu.sync_copy(ones_hbm, r0_v)
        zero_own()
        plsc.subcore_barrier()

        def count_rel(dst2_hbm):
            def cblock(blk, carry):
                base = s * NCH + blk * NCB
                pltpu.sync_copy(dst2_hbm.at[pl.ds(base, NCB)], idxd_v)
                cps = [pltpu.async_copy(
                           r0_v, acc_sh.at[idxd_v.at[j]], ssem[j % 3],
                           add=True)
                       for j in range(NCB)]
                for cp in cps:
                    cp.wait()
                return carry
            lax.fori_loop(0, NBLK, cblock, 0)

        @pl.when(c == 0)
        def _():
            count_rel(dst_p2_hbm)

        @pl.when(c == 1)
        def _():
            count_rel(dst_r2_hbm)

        plsc.subcore_barrier()
        drain_own(cnt3_hbm.at[c])

    def one_relation(src2_hbm, dst2_hbm, tab3_hbm, out3_hbm):
        zero_own()
        plsc.subcore_barrier()
        tab2 = tab3_hbm.at[c]

        def block(blk, carry):
            base = s * NCH + blk * NCB
            pltpu.sync_copy(src2_hbm.at[pl.ds(base, NCB)], idxs_v)
            pltpu.sync_copy(dst2_hbm.at[pl.ds(base, NCB)], idxd_v)
            gd = {}
            sd = {}
            for j in range(3):
                gd[j] = pltpu.async_copy(
                    tab2.at[idxs_v.at[j]], rows[j], gsem[j])
            for j in range(NCB):
                gd[j].wait()
                sd[j] = pltpu.async_copy(
                    rows[j % 3], acc_sh.at[idxd_v.at[j]], ssem[j % 3],
                    add=True)
                nj = j + 3
                if nj < NCB:
                    sd[j].wait()
                    gd[nj] = pltpu.async_copy(
                        tab2.at[idxs_v.at[nj]], rows[nj % 3], gsem[nj % 3])
            for j in range(NCB - 3, NCB):
                sd[j].wait()
            return carry

        lax.fori_loop(0, NBLK, block, 0)
        plsc.subcore_barrier()
        drain_own(out3_hbm.at[c])

    one_relation(src_r2_hbm, dst_p2_hbm, tabr3_hbm, aggp3_hbm)
    one_relation(src_p2_hbm, dst_r2_hbm, tabp3_hbm, aggr3_hbm)


_sc_call = functools.partial(
    pl.kernel,
    out_type=[jax.ShapeDtypeStruct((2, PAD, HH), jnp.float32)] * 3,
    mesh=_mesh,
    scratch_types=[
        pltpu.VMEM((NCB, EC), jnp.int32),
        pltpu.VMEM((NCB, EC), jnp.int32),
        pltpu.VMEM((EC, HH), jnp.float32),
        pltpu.VMEM((EC, HH), jnp.float32),
        pltpu.VMEM((EC, HH), jnp.float32),
        pltpu.VMEM((DR, HH), jnp.float32),
        pltpu.VMEM((16,), jnp.int32),
        pltpu.VMEM_SHARED((PAD, HH), jnp.float32),
        pltpu.SemaphoreType.DMA,
        pltpu.SemaphoreType.DMA,
        pltpu.SemaphoreType.DMA,
        pltpu.SemaphoreType.DMA,
        pltpu.SemaphoreType.DMA,
        pltpu.SemaphoreType.DMA,
        pltpu.SemaphoreType.DMA,
    ],
    compiler_params=pltpu.CompilerParams(use_tc_tiling_on_sc=False,
                                        needs_layout_passes=False),
)(_sc_body)


def _proj_body(xr_ref, xp_ref, wr_ref, br_ref, wp_ref, bp_ref,
               hr_ref, hp_ref):
    hr = jnp.dot(xr_ref[...], wr_ref[...],
                 preferred_element_type=jnp.float32) + br_ref[...]
    hp = jnp.dot(xp_ref[...], wp_ref[...],
                 preferred_element_type=jnp.float32) + bp_ref[...]
    hr_ref[0] = hr[:, :HH]
    hr_ref[1] = hr[:, HH:]
    hp_ref[0] = hp[:, :HH]
    hp_ref[1] = hp[:, HH:]


def _proj(x_review, x_product, W_review, b_review, W_product, b_product):
    blk = lambda: pl.BlockSpec((BB, DIN), lambda i: (i, 0))
    full = lambda shape: pl.BlockSpec(shape, lambda i: tuple(0 for _ in shape))
    outb = lambda: pl.BlockSpec((2, BB, HH), lambda i: (0, i, 0))
    return pl.pallas_call(
        _proj_body,
        grid=(NB,),
        in_specs=[blk(), blk(), full((DIN, H)), full((1, H)),
                  full((DIN, H)), full((1, H))],
        out_specs=[outb(), outb()],
        out_shape=[jax.ShapeDtypeStruct((2, NR, HH), jnp.float32)] * 2,
    )(x_review, x_product, W_review, b_review.reshape(1, H),
      W_product, b_product.reshape(1, H))


def _combine_body(split_out, relu,
                  hr_ref, hp_ref, ap_ref, ar_ref, cnt_ref,
                  wroot_ref, wrel_ref, b_ref,
                  *out_refs):
    inv_p = 1.0 / jnp.maximum(cnt_ref[0][:, 0:1], 1.0)
    inv_r = 1.0 / jnp.maximum(cnt_ref[1][:, 0:1], 1.0)
    h_rev = jnp.concatenate([hr_ref[0], hr_ref[1]], axis=1)
    h_prod = jnp.concatenate([hp_ref[0], hp_ref[1]], axis=1)
    agg_p = jnp.concatenate([ap_ref[0], ap_ref[1]], axis=1) * inv_p
    agg_r = jnp.concatenate([ar_ref[0], ar_ref[1]], axis=1) * inv_r
    wroot = wroot_ref[...]
    b = b_ref[...]
    out_rev = (jnp.dot(h_rev, wroot, preferred_element_type=jnp.float32) + b
               + jnp.dot(agg_r, wrel_ref[1], preferred_element_type=jnp.float32))
    out_prod = (jnp.dot(h_prod, wroot, preferred_element_type=jnp.float32) + b
                + jnp.dot(agg_p, wrel_ref[0], preferred_element_type=jnp.float32))
    if relu:
        out_rev = jnp.maximum(out_rev, 0.0)
        out_prod = jnp.maximum(out_prod, 0.0)
    if split_out:
        out_refs[0][0] = out_rev[:, :HH]
        out_refs[0][1] = out_rev[:, HH:]
        out_refs[1][0] = out_prod[:, :HH]
        out_refs[1][1] = out_prod[:, HH:]
    else:
        out_refs[0][0] = out_rev
        out_refs[0][1] = out_prod


def _combine(hr3, hp3, aggp3, aggr3, cnt3, w_rel, w_root, b,
             split_out, relu):
    dout = w_root.shape[1]
    pblk = lambda w: pl.BlockSpec((2, BB, w), lambda i: (0, i, 0))
    full = lambda shape: pl.BlockSpec(shape, lambda i: tuple(0 for _ in shape))
    if split_out:
        out_specs = [pblk(HH), pblk(HH)]
        out_shape = [jax.ShapeDtypeStruct((2, NR, HH), jnp.float32)] * 2
    else:
        out_specs = [pl.BlockSpec((2, BB, dout), lambda i: (0, i, 0))]
        out_shape = [jax.ShapeDtypeStruct((2, NR, dout), jnp.float32)]
    return pl.pallas_call(
        functools.partial(_combine_body, split_out, relu),
        grid=(NB,),
        in_specs=[pblk(HH)] * 5
                 + [full((H, dout)), full((2, H, dout)), full((1, dout))],
        out_specs=out_specs,
        out_shape=out_shape,
    )(hr3, hp3, aggp3, aggr3, cnt3, w_root, w_rel, b.reshape(1, dout))


def kernel(x_review, x_product, edge_index_r2p, edge_index_p2r,
           W_review, b_review, W_product, b_product,
           conv1_w_rel, conv1_w_root, conv1_b,
           conv2_w_rel, conv2_w_root, conv2_b):
    src_r2 = edge_index_r2p[0].reshape(E // EC, EC)
    dst_p2 = (edge_index_r2p[1] - NR).reshape(E // EC, EC)
    src_p2 = (edge_index_p2r[0] - NR).reshape(E // EC, EC)
    dst_r2 = edge_index_p2r[1].reshape(E // EC, EC)
    ones = jnp.ones((EC, HH), jnp.float32)
    zrows = jnp.zeros((DR, HH), jnp.float32)
    flag1 = jnp.ones((16,), jnp.int32)
    flag0 = jnp.zeros((16,), jnp.int32)

    hr3, hp3 = _proj(x_review, x_product, W_review, b_review,
                     W_product, b_product)
    cnt3, aggp3, aggr3 = _sc_call(hr3, hp3, src_r2, dst_p2, src_p2, dst_r2,
                                  ones, zrows, flag1)
    hr3b, hp3b = _combine(hr3, hp3, aggp3, aggr3, cnt3,
                          conv1_w_rel, conv1_w_root, conv1_b,
                          split_out=True, relu=True)
    _, aggp3b, aggr3b = _sc_call(hr3b, hp3b, src_r2, dst_p2, src_p2, dst_r2,
                                 ones, zrows, flag0)
    (out,) = _combine(hr3b, hp3b, aggp3b, aggr3b, cnt3,
                      conv2_w_rel, conv2_w_root, conv2_b,
                      split_out=False, relu=False)
    return out.reshape(2 * NR, OUT)


# speedup vs baseline: 6.6438x; 1.1569x over previous
import functools

import jax
import jax.numpy as jnp
from jax import lax
from jax.experimental import pallas as pl
from jax.experimental.pallas import tpu as pltpu
from jax.experimental.pallas import tpu_sc as plsc

NR = 50000
DIN = 128
H = 64
HH = 32
OUT = 32
E = 400000
NT = 16
TPR = 3136
PAD = NT * TPR
EPT = E // NT
EC = 200
NCH = EPT // EC
NCB = 25
NBLK = NCH // NCB
DR = 28
NDR = TPR // DR
BB = 2000
NB = NR // BB

_mesh = plsc.VectorSubcoreMesh(core_axis_name="c", subcore_axis_name="s")


def _sc_body(tabr3_hbm, tabp3_hbm,
             src_r2_hbm, dst_p2_hbm, src_p2_hbm, dst_r2_hbm,
             ones_hbm, zrows_hbm, flag_hbm,
             cnt3_hbm, aggp3_hbm, aggr3_hbm,
             idxs_v, idxd_v, r0_v, r1_v, r2_v, zstage_v, flag_v, acc_sh,
             g0, g1, g2, s0, s1, s2, zsem):
    c = lax.axis_index("c")
    s = lax.axis_index("s")
    rows = [r0_v, r1_v, r2_v]
    gsem = [g0, g1, g2]
    ssem = [s0, s1, s2]
    pltpu.sync_copy(zrows_hbm, zstage_v)
    pltpu.sync_copy(flag_hbm, flag_v)
    do_counts = jnp.max(flag_v[...])

    def zero_own():
        cps = [pltpu.async_copy(
                   zstage_v, acc_sh.at[pl.ds(s * TPR + j * DR, DR)], zsem)
               for j in range(NDR)]
        for cp in cps:
            cp.wait()

    def drain_own(out2):
        cps = [pltpu.async_copy(
                   acc_sh.at[pl.ds(s * TPR + j * DR, DR)],
                   out2.at[pl.ds(s * TPR + j * DR, DR)], zsem)
               for j in range(NDR)]
        for cp in cps:
            cp.wait()

    @pl.when(do_counts > 0)
    def _():
        pltpu.sync_copy(ones_hbm, r0_v)
        zero_own()
        plsc.subcore_barrier()

        def count_rel(dst2_hbm):
            def cblock(blk, carry):
                base = s * NCH + blk * NCB
                pltpu.sync_copy(dst2_hbm.at[pl.ds(base, NCB)], idxd_v)
                cps = [pltpu.async_copy(
                           r0_v, acc_sh.at[idxd_v.at[j]], ssem[j % 3],
                           add=True)
                       for j in range(NCB)]
                for cp in cps:
                    cp.wait()
                return carry
            lax.fori_loop(0, NBLK, cblock, 0)

        @pl.when(c == 0)
        def _():
            count_rel(dst_p2_hbm)

        @pl.when(c == 1)
        def _():
            count_rel(dst_r2_hbm)

        plsc.subcore_barrier()
        drain_own(cnt3_hbm.at[c])

    def one_relation(src2_hbm, dst2_hbm, tab3_hbm, out3_hbm):
        zero_own()
        plsc.subcore_barrier()
        tab2 = tab3_hbm.at[c]

        def block(blk, carry):
            base = s * NCH + blk * NCB
            pltpu.sync_copy(src2_hbm.at[pl.ds(base, NCB)], idxs_v)
            pltpu.sync_copy(dst2_hbm.at[pl.ds(base, NCB)], idxd_v)
            gd = {}
            sd = {}
            for j in range(3):
                gd[j] = pltpu.async_copy(
                    tab2.at[idxs_v.at[j]], rows[j], gsem[j])
            for j in range(NCB):
                gd[j].wait()
                sd[j] = pltpu.async_copy(
                    rows[j % 3], acc_sh.at[idxd_v.at[j]], ssem[j % 3],
                    add=True)
                nj = j + 3
                if nj < NCB:
                    sd[j].wait()
                    gd[nj] = pltpu.async_copy(
                        tab2.at[idxs_v.at[nj]], rows[nj % 3], gsem[nj % 3])
            for j in range(NCB - 3, NCB):
                sd[j].wait()
            return carry

        lax.fori_loop(0, NBLK, block, 0)
        plsc.subcore_barrier()
        drain_own(out3_hbm.at[c])

    one_relation(src_r2_hbm, dst_p2_hbm, tabr3_hbm, aggp3_hbm)
    one_relation(src_p2_hbm, dst_r2_hbm, tabp3_hbm, aggr3_hbm)


_sc_call = functools.partial(
    pl.kernel,
    out_type=[jax.ShapeDtypeStruct((2, PAD, HH), jnp.float32)] * 3,
    mesh=_mesh,
    scratch_types=[
        pltpu.VMEM((NCB, EC), jnp.int32),
        pltpu.VMEM((NCB, EC), jnp.int32),
        pltpu.VMEM((EC, HH), jnp.float32),
        pltpu.VMEM((EC, HH), jnp.float32),
        pltpu.VMEM((EC, HH), jnp.float32),
        pltpu.VMEM((DR, HH), jnp.float32),
        pltpu.VMEM((16,), jnp.int32),
        pltpu.VMEM_SHARED((PAD, HH), jnp.float32),
        pltpu.SemaphoreType.DMA,
        pltpu.SemaphoreType.DMA,
        pltpu.SemaphoreType.DMA,
        pltpu.SemaphoreType.DMA,
        pltpu.SemaphoreType.DMA,
        pltpu.SemaphoreType.DMA,
        pltpu.SemaphoreType.DMA,
    ],
    compiler_params=pltpu.CompilerParams(use_tc_tiling_on_sc=False,
                                        needs_layout_passes=False),
)(_sc_body)


def _proj_body(xr_ref, xp_ref, wr_ref, br_ref, wp_ref, bp_ref,
               hr_ref, hp_ref):
    hr = jnp.dot(xr_ref[...], wr_ref[...],
                 preferred_element_type=jnp.float32) + br_ref[...]
    hp = jnp.dot(xp_ref[...], wp_ref[...],
                 preferred_element_type=jnp.float32) + bp_ref[...]
    hr_ref[0] = hr[:, :HH]
    hr_ref[1] = hr[:, HH:]
    hp_ref[0] = hp[:, :HH]
    hp_ref[1] = hp[:, HH:]


def _proj(x_review, x_product, W_review, b_review, W_product, b_product):
    blk = lambda: pl.BlockSpec((BB, DIN), lambda i: (i, 0))
    full = lambda shape: pl.BlockSpec(shape, lambda i: tuple(0 for _ in shape))
    outb = lambda: pl.BlockSpec((2, BB, HH), lambda i: (0, i, 0))
    return pl.pallas_call(
        _proj_body,
        grid=(NB,),
        in_specs=[blk(), blk(), full((DIN, H)), full((1, H)),
                  full((DIN, H)), full((1, H))],
        out_specs=[outb(), outb()],
        out_shape=[jax.ShapeDtypeStruct((2, NR, HH), jnp.float32)] * 2,
    )(x_review, x_product, W_review, b_review.reshape(1, H),
      W_product, b_product.reshape(1, H))


def _combine_body(split_out, relu,
                  hr_ref, hp_ref, ap_ref, ar_ref, cnt_ref,
                  wroot_ref, wrel_ref, b_ref,
                  *out_refs):
    inv_p = 1.0 / jnp.maximum(cnt_ref[0][:, 0:1], 1.0)
    inv_r = 1.0 / jnp.maximum(cnt_ref[1][:, 0:1], 1.0)
    h_rev = jnp.concatenate([hr_ref[0], hr_ref[1]], axis=1)
    h_prod = jnp.concatenate([hp_ref[0], hp_ref[1]], axis=1)
    agg_p = jnp.concatenate([ap_ref[0], ap_ref[1]], axis=1) * inv_p
    agg_r = jnp.concatenate([ar_ref[0], ar_ref[1]], axis=1) * inv_r
    wroot = wroot_ref[...]
    b = b_ref[...]
    out_rev = (jnp.dot(h_rev, wroot, preferred_element_type=jnp.float32) + b
               + jnp.dot(agg_r, wrel_ref[1], preferred_element_type=jnp.float32))
    out_prod = (jnp.dot(h_prod, wroot, preferred_element_type=jnp.float32) + b
                + jnp.dot(agg_p, wrel_ref[0], preferred_element_type=jnp.float32))
    if relu:
        out_rev = jnp.maximum(out_rev, 0.0)
        out_prod = jnp.maximum(out_prod, 0.0)
    if split_out:
        out_refs[0][0] = out_rev[:, :HH]
        out_refs[0][1] = out_rev[:, HH:]
        out_refs[1][0] = out_prod[:, :HH]
        out_refs[1][1] = out_prod[:, HH:]
    else:
        out_refs[0][0] = out_rev
        out_refs[0][1] = out_prod


def _combine(hr3, hp3, aggp3, aggr3, cnt3, w_rel, w_root, b,
             split_out, relu):
    dout = w_root.shape[1]
    pblk = lambda w: pl.BlockSpec((2, BB, w), lambda i: (0, i, 0))
    full = lambda shape: pl.BlockSpec(shape, lambda i: tuple(0 for _ in shape))
    if split_out:
        out_specs = [pblk(HH), pblk(HH)]
        out_shape = [jax.ShapeDtypeStruct((2, NR, HH), jnp.float32)] * 2
    else:
        out_specs = [pl.BlockSpec((2, BB, dout), lambda i: (0, i, 0))]
        out_shape = [jax.ShapeDtypeStruct((2, NR, dout), jnp.float32)]
    return pl.pallas_call(
        functools.partial(_combine_body, split_out, relu),
        grid=(NB,),
        in_specs=[pblk(HH)] * 5
                 + [full((H, dout)), full((2, H, dout)), full((1, dout))],
        out_specs=out_specs,
        out_shape=out_shape,
    )(hr3, hp3, aggp3, aggr3, cnt3, w_root, w_rel, b.reshape(1, dout))


def kernel(x_review, x_product, edge_index_r2p, edge_index_p2r,
           W_review, b_review, W_product, b_product,
           conv1_w_rel, conv1_w_root, conv1_b,
           conv2_w_rel, conv2_w_root, conv2_b):
    src_r2 = edge_index_r2p[0].reshape(E // EC, EC)
    dst_p2 = (edge_index_r2p[1] - NR).reshape(E // EC, EC)
    src_p2 = (edge_index_p2r[0] - NR).reshape(E // EC, EC)
    dst_r2 = edge_index_p2r[1].reshape(E // EC, EC)
    ones = jnp.ones((EC, HH), jnp.float32)
    zrows = jnp.zeros((DR, HH), jnp.float32)
    flag1 = jnp.ones((16,), jnp.int32)
    flag0 = jnp.zeros((16,), jnp.int32)

    hr3, hp3 = _proj(x_review, x_product, W_review, b_review,
                     W_product, b_product)
    cnt3, aggp3, aggr3 = _sc_call(hr3, hp3, src_r2, dst_p2, src_p2, dst_r2,
                                  ones, zrows, flag1)
    hr3b, hp3b = _combine(hr3, hp3, aggp3, aggr3, cnt3,
                          conv1_w_rel, conv1_w_root, conv1_b,
                          split_out=True, relu=True)
    _, aggp3b, aggr3b = _sc_call(hr3b, hp3b, src_r2, dst_p2, src_p2, dst_r2,
                                 ones, zrows, flag0)
    (out,) = _combine(hr3b, hp3b, aggp3b, aggr3b, cnt3,
                      conv2_w_rel, conv2_w_root, conv2_b,
                      split_out=False, relu=False)
    return out.reshape(2 * NR, OUT)
